# Initial kernel scaffold; baseline (speedup 1.0000x reference)
import jax
import jax.numpy as jnp
from jax.experimental import pallas as pl


def kernel(x, table):
    raise NotImplementedError("write your pallas kernel here")


# speedup vs baseline: 1.1421x; 1.1421x over previous
import functools

import jax
import jax.numpy as jnp
from jax import lax
from jax.experimental import pallas as pl
from jax.experimental.pallas import tpu as pltpu
from jax.experimental.pallas import tpu_sc as plsc

B_TOTAL = 4096 * 50
D = 128
NC = 2
NS = 16
NW = NC * NS
B_PER_W = B_TOTAL // NW
CHUNK = 128
NCHUNKS = B_PER_W // CHUNK


def _gather_body(x_hbm, table_hbm, out_hbm, idx_v, rows_v, gsem):
    wid = lax.axis_index("s") * NC + lax.axis_index("c")
    base = wid * B_PER_W
    pltpu.sync_copy(x_hbm.at[wid], idx_v)

    def step(j, carry):
        pltpu.async_copy(table_hbm.at[idx_v.at[j]], rows_v, gsem).wait()
        pltpu.sync_copy(rows_v, out_hbm.at[pl.ds(base + j * CHUNK, CHUNK)])
        return carry

    lax.fori_loop(0, NCHUNKS, step, 0)


@jax.jit
def _gather(x_flat, table):
    mesh = plsc.VectorSubcoreMesh(core_axis_name="c", subcore_axis_name="s")
    f = functools.partial(
        pl.kernel,
        mesh=mesh,
        out_type=jax.ShapeDtypeStruct((B_TOTAL, D), jnp.float32),
        scratch_types=[
            pltpu.VMEM((NCHUNKS, CHUNK), jnp.int32),
            pltpu.VMEM((CHUNK, D), jnp.float32),
            pltpu.SemaphoreType.DMA,
        ],
    )(_gather_body)
    return f(x_flat, table)


def kernel(x, table):
    x_flat = x.reshape(NW, NCHUNKS, CHUNK).astype(jnp.int32)
    out = _gather(x_flat, table)
    return out.reshape(x.shape[0], x.shape[1], D)


# speedup vs baseline: 1.2028x; 1.0531x over previous
import functools

import jax
import jax.numpy as jnp
from jax import lax
from jax.experimental import pallas as pl
from jax.experimental.pallas import tpu as pltpu
from jax.experimental.pallas import tpu_sc as plsc

B_TOTAL = 4096 * 50
D = 128
NC = 2
NS = 16
NW = NC * NS
B_PER_W = B_TOTAL // NW
CHUNK = 128
NCHUNKS = B_PER_W // CHUNK


def _gather_body(x_hbm, table_hbm, out_hbm,
                 idx_v, rows0, rows1, gs0, gs1, ws0, ws1):
    wid = lax.axis_index("s") * NC + lax.axis_index("c")
    base = wid * B_PER_W
    rows = (rows0, rows1)
    gsem = (gs0, gs1)
    wsem = (ws0, ws1)

    pltpu.sync_copy(x_hbm.at[wid], idx_v)

    def start_gather(c, b):
        pltpu.async_copy(table_hbm.at[idx_v.at[c]], rows[b], gsem[b])

    def wait_gather(b):
        pltpu.make_async_copy(table_hbm.at[pl.ds(0, CHUNK)], rows[b],
                              gsem[b]).wait()

    def start_wb(c, b):
        pltpu.async_copy(rows[b], out_hbm.at[pl.ds(base + c * CHUNK, CHUNK)],
                         wsem[b])

    def wait_wb(b):
        pltpu.make_async_copy(rows[b], out_hbm.at[pl.ds(base, CHUNK)],
                              wsem[b]).wait()

    start_gather(0, 0)
    wait_gather(0)
    start_gather(1, 1)
    start_wb(0, 0)

    def outer(t, carry):
        c0 = 1 + 2 * t
        for k, b in enumerate((1, 0)):
            c = c0 + k
            wait_gather(b)
            wait_wb(1 - b)
            start_gather(c + 1, 1 - b)
            start_wb(c, b)
        return carry

    lax.fori_loop(0, (NCHUNKS - 2) // 2, outer, 0)

    wait_gather(1)
    wait_wb(0)
    start_wb(NCHUNKS - 1, 1)
    wait_wb(1)


@jax.jit
def _gather(x_flat, table):
    mesh = plsc.VectorSubcoreMesh(core_axis_name="c", subcore_axis_name="s")
    f = functools.partial(
        pl.kernel,
        mesh=mesh,
        out_type=jax.ShapeDtypeStruct((B_TOTAL, D), jnp.float32),
        scratch_types=[
            pltpu.VMEM((NCHUNKS, CHUNK), jnp.int32),
            pltpu.VMEM((CHUNK, D), jnp.float32),
            pltpu.VMEM((CHUNK, D), jnp.float32),
            pltpu.SemaphoreType.DMA,
            pltpu.SemaphoreType.DMA,
            pltpu.SemaphoreType.DMA,
            pltpu.SemaphoreType.DMA,
        ],
    )(_gather_body)
    return f(x_flat, table)


def kernel(x, table):
    x_flat = x.reshape(NW, NCHUNKS, CHUNK).astype(jnp.int32)
    out = _gather(x_flat, table)
    return out.reshape(x.shape[0], x.shape[1], D)


# speedup vs baseline: 1.5981x; 1.3287x over previous
import functools

import jax
import jax.numpy as jnp
from jax import lax
from jax.experimental import pallas as pl
from jax.experimental.pallas import tpu as pltpu
from jax.experimental.pallas import tpu_sc as plsc

NI = 4096
NJ = 50
D = 128
NC = 2
NS = 16
NW = NC * NS
NI_PER_W = NI // NW


def _gather_body(x_hbm, table_hbm, out_hbm,
                 idx_v, rows0, rows1, gs0, gs1, ws0, ws1):
    wid = lax.axis_index("s") * NC + lax.axis_index("c")
    i0 = wid * NI_PER_W
    rows = (rows0, rows1)
    gsem = (gs0, gs1)
    wsem = (ws0, ws1)

    pltpu.sync_copy(x_hbm.at[pl.ds(i0, NI_PER_W)], idx_v)

    def start_gather(c, b):
        pltpu.async_copy(table_hbm.at[idx_v.at[c]], rows[b], gsem[b])

    def wait_gather(b):
        pltpu.make_async_copy(table_hbm.at[idx_v.at[0]], rows[b],
                              gsem[b]).wait()

    def start_wb(c, b):
        pltpu.async_copy(rows[b], out_hbm.at[i0 + c], wsem[b])

    def wait_wb(b):
        pltpu.make_async_copy(rows[b], out_hbm.at[0], wsem[b]).wait()

    start_gather(0, 0)
    wait_gather(0)
    start_gather(1, 1)
    start_wb(0, 0)

    def outer(t, carry):
        c0 = 1 + 2 * t
        for k, b in enumerate((1, 0)):
            c = c0 + k
            wait_gather(b)
            wait_wb(1 - b)
            start_gather(c + 1, 1 - b)
            start_wb(c, b)
        return carry

    lax.fori_loop(0, (NI_PER_W - 2) // 2, outer, 0)

    wait_gather(1)
    wait_wb(0)
    start_wb(NI_PER_W - 1, 1)
    wait_wb(1)


@jax.jit
def _gather(x, table):
    mesh = plsc.VectorSubcoreMesh(core_axis_name="c", subcore_axis_name="s")
    f = functools.partial(
        pl.kernel,
        mesh=mesh,
        out_type=jax.ShapeDtypeStruct((NI, NJ, D), jnp.float32),
        scratch_types=[
            pltpu.VMEM((NI_PER_W, NJ), jnp.int32),
            pltpu.VMEM((NJ, D), jnp.float32),
            pltpu.VMEM((NJ, D), jnp.float32),
            pltpu.SemaphoreType.DMA,
            pltpu.SemaphoreType.DMA,
            pltpu.SemaphoreType.DMA,
            pltpu.SemaphoreType.DMA,
        ],
        compiler_params=pltpu.CompilerParams(use_tc_tiling_on_sc=True),
    )(_gather_body)
    return f(x, table)


def kernel(x, table):
    return _gather(x.astype(jnp.int32), table)


# speedup vs baseline: 2.2432x; 1.4037x over previous
import functools

import jax
import jax.numpy as jnp
from jax import lax
from jax.experimental import pallas as pl
from jax.experimental.pallas import tpu as pltpu
from jax.experimental.pallas import tpu_sc as plsc

NI = 4096
NJ = 50
D = 128
NC = 2
NS = 16
NW = NC * NS
NI_PER_W = NI // NW
NBUF = 4


def _gather_body(x_hbm, table_hbm, out_hbm, idx_v,
                 rows0, rows1, rows2, rows3,
                 gs0, gs1, gs2, gs3, ws0, ws1, ws2, ws3):
    wid = lax.axis_index("s") * NC + lax.axis_index("c")
    i0 = wid * NI_PER_W
    rows = (rows0, rows1, rows2, rows3)
    gsem = (gs0, gs1, gs2, gs3)
    wsem = (ws0, ws1, ws2, ws3)

    pltpu.sync_copy(x_hbm.at[pl.ds(i0, NI_PER_W)], idx_v)

    def start_gather(c, b):
        pltpu.async_copy(table_hbm.at[idx_v.at[c]], rows[b], gsem[b])

    def wait_gather(b):
        pltpu.make_async_copy(table_hbm.at[idx_v.at[0]], rows[b],
                              gsem[b]).wait()

    def start_wb(c, b):
        pltpu.async_copy(rows[b], out_hbm.at[i0 + c], wsem[b])

    def wait_wb(b):
        pltpu.make_async_copy(rows[b], out_hbm.at[0], wsem[b]).wait()

    start_gather(0, 0)
    start_gather(1, 1)
    start_gather(2, 2)

    wait_gather(0)
    start_wb(0, 0)
    start_gather(3, 3)

    def outer(t, carry):
        c0 = 1 + NBUF * t
        for k in range(NBUF):
            c = c0 + k
            b = (1 + k) % NBUF
            bprev = k
            wait_gather(b)
            start_wb(c, b)
            wait_wb(bprev)
            start_gather(c + 3, bprev)
        return carry

    lax.fori_loop(0, (NI_PER_W - NBUF) // NBUF, outer, 0)

    for k in range(3):
        c = NI_PER_W - 3 + k
        b = (1 + k) % NBUF
        wait_gather(b)
        start_wb(c, b)
        wait_wb(k)
    wait_wb(3)


@jax.jit
def _gather(x, table):
    mesh = plsc.VectorSubcoreMesh(core_axis_name="c", subcore_axis_name="s")
    f = functools.partial(
        pl.kernel,
        mesh=mesh,
        out_type=jax.ShapeDtypeStruct((NI, NJ, D), jnp.float32),
        scratch_types=[
            pltpu.VMEM((NI_PER_W, NJ), jnp.int32),
            pltpu.VMEM((NJ, D), jnp.float32),
            pltpu.VMEM((NJ, D), jnp.float32),
            pltpu.VMEM((NJ, D), jnp.float32),
            pltpu.VMEM((NJ, D), jnp.float32),
            pltpu.SemaphoreType.DMA,
            pltpu.SemaphoreType.DMA,
            pltpu.SemaphoreType.DMA,
            pltpu.SemaphoreType.DMA,
            pltpu.SemaphoreType.DMA,
            pltpu.SemaphoreType.DMA,
            pltpu.SemaphoreType.DMA,
            pltpu.SemaphoreType.DMA,
        ],
        compiler_params=pltpu.CompilerParams(use_tc_tiling_on_sc=True),
    )(_gather_body)
    return f(x, table)


def kernel(x, table):
    return _gather(x.astype(jnp.int32), table)


# speedup vs baseline: 2.2820x; 1.0173x over previous
import functools

import jax
import jax.numpy as jnp
from jax import lax
from jax.experimental import pallas as pl
from jax.experimental.pallas import tpu as pltpu
from jax.experimental.pallas import tpu_sc as plsc

NI = 4096
NJ = 50
D = 128
NC = 2
NS = 16
NW = NC * NS
NI_PER_W = NI // NW
NBUF = 8

assert (NI_PER_W - NBUF) % NBUF == 0


def _gather_body(x_hbm, table_hbm, out_hbm, idx_v, rows, gsem, wsem):
    wid = lax.axis_index("s") * NC + lax.axis_index("c")
    i0 = wid * NI_PER_W

    pltpu.sync_copy(x_hbm.at[pl.ds(i0, NI_PER_W)], idx_v)

    def start_gather(c, b):
        pltpu.async_copy(table_hbm.at[idx_v.at[c]], rows[b], gsem[b])

    def wait_gather(b):
        pltpu.make_async_copy(table_hbm.at[idx_v.at[0]], rows[b],
                              gsem[b]).wait()

    def start_wb(c, b):
        pltpu.async_copy(rows[b], out_hbm.at[i0 + c], wsem[b])

    def wait_wb(b):
        pltpu.make_async_copy(rows[b], out_hbm.at[0], wsem[b]).wait()

    for b in range(NBUF - 1):
        start_gather(b, b)

    wait_gather(0)
    start_wb(0, 0)
    start_gather(NBUF - 1, NBUF - 1)

    def outer(t, carry):
        c0 = 1 + NBUF * t
        for k in range(NBUF):
            c = c0 + k
            b = (1 + k) % NBUF
            bprev = k
            wait_gather(b)
            start_wb(c, b)
            wait_wb(bprev)
            start_gather(c + NBUF - 1, bprev)
        return carry

    lax.fori_loop(0, (NI_PER_W - NBUF) // NBUF, outer, 0)

    for k in range(NBUF - 1):
        c = NI_PER_W - (NBUF - 1) + k
        b = c % NBUF
        wait_gather(b)
        start_wb(c, b)
        wait_wb((c - 1) % NBUF)
    wait_wb((NI_PER_W - 1) % NBUF)


@jax.jit
def _gather(x, table):
    mesh = plsc.VectorSubcoreMesh(core_axis_name="c", subcore_axis_name="s")
    f = functools.partial(
        pl.kernel,
        mesh=mesh,
        out_type=jax.ShapeDtypeStruct((NI, NJ, D), jnp.float32),
        scratch_types=[
            pltpu.VMEM((NI_PER_W, NJ), jnp.int32),
            [pltpu.VMEM((NJ, D), jnp.float32)] * NBUF,
            [pltpu.SemaphoreType.DMA] * NBUF,
            [pltpu.SemaphoreType.DMA] * NBUF,
        ],
        compiler_params=pltpu.CompilerParams(use_tc_tiling_on_sc=True),
    )(_gather_body)
    return f(x, table)


def kernel(x, table):
    return _gather(x.astype(jnp.int32), table)


# speedup vs baseline: 4.1056x; 1.7992x over previous
import functools

import jax
import jax.numpy as jnp
from jax import lax
from jax.experimental import pallas as pl
from jax.experimental.pallas import tpu as pltpu
from jax.experimental.pallas import tpu_sc as plsc

NI = 4096
NJ = 50
D = 128
NC = 2
NS = 16
NW = NC * NS
CHUNK = NI // NW
NBUF = 5

assert (NJ - NBUF) % NBUF == 0


def _gather_body(xt_hbm, table_hbm, out_hbm, idx_v, rows, gsem, wsem):
    wid = lax.axis_index("s") * NC + lax.axis_index("c")
    i0 = wid * CHUNK

    pltpu.sync_copy(xt_hbm.at[:, pl.ds(i0, CHUNK)], idx_v)

    def start_gather(c, b):
        pltpu.async_copy(table_hbm.at[idx_v.at[c]], rows[b], gsem[b])

    def wait_gather(b):
        pltpu.make_async_copy(table_hbm.at[idx_v.at[0]], rows[b],
                              gsem[b]).wait()

    def start_wb(c, b):
        pltpu.async_copy(rows[b], out_hbm.at[c, pl.ds(i0, CHUNK)], wsem[b])

    def wait_wb(b):
        pltpu.make_async_copy(rows[b], out_hbm.at[0, pl.ds(0, CHUNK)],
                              wsem[b]).wait()

    for b in range(NBUF - 1):
        start_gather(b, b)

    wait_gather(0)
    start_wb(0, 0)
    start_gather(NBUF - 1, NBUF - 1)

    def outer(t, carry):
        c0 = 1 + NBUF * t
        for k in range(NBUF):
            c = c0 + k
            b = (1 + k) % NBUF
            bprev = k
            wait_gather(b)
            start_wb(c, b)
            wait_wb(bprev)
            start_gather(c + NBUF - 1, bprev)
        return carry

    lax.fori_loop(0, (NJ - NBUF) // NBUF, outer, 0)

    for k in range(NBUF - 1):
        c = NJ - (NBUF - 1) + k
        b = c % NBUF
        wait_gather(b)
        start_wb(c, b)
        wait_wb((c - 1) % NBUF)
    wait_wb((NJ - 1) % NBUF)


@jax.jit
def _gather(xt, table):
    mesh = plsc.VectorSubcoreMesh(core_axis_name="c", subcore_axis_name="s")
    f = functools.partial(
        pl.kernel,
        mesh=mesh,
        out_type=jax.ShapeDtypeStruct((NJ, NI, D), jnp.float32),
        scratch_types=[
            pltpu.VMEM((NJ, CHUNK), jnp.int32),
            [pltpu.VMEM((CHUNK, D), jnp.float32)] * NBUF,
            [pltpu.SemaphoreType.DMA] * NBUF,
            [pltpu.SemaphoreType.DMA] * NBUF,
        ],
        compiler_params=pltpu.CompilerParams(use_tc_tiling_on_sc=True),
    )(_gather_body)
    return f(xt, table)


def kernel(x, table):
    out_t = _gather(x.T.astype(jnp.int32), table)
    return out_t.transpose(1, 0, 2)
